# baseline (speedup 1.0000x reference)
import functools

import jax
import jax.numpy as jnp
from jax import lax
from jax.experimental import pallas as pl
from jax.experimental.pallas import tpu as pltpu
from jax.experimental.pallas import tpu_sc as plsc

NC, NS = 2, 16
NW = NC * NS
LANES = 16


def _sc_embed(table, idx, pos):
    B, S = idx.shape
    D = table.shape[1]
    b_per_w = B // NW
    mesh = plsc.VectorSubcoreMesh(core_axis_name="c", subcore_axis_name="s")

    @functools.partial(
        pl.kernel,
        out_type=jax.ShapeDtypeStruct((B, S, D), jnp.float32),
        mesh=mesh,
        scratch_types=[
            pltpu.VMEM((b_per_w, S), jnp.int32),
            pltpu.VMEM((S, D), jnp.float32),
            pltpu.VMEM((2, S, D), jnp.float32),
            pltpu.VMEM((S, D), jnp.float32),
            pltpu.SemaphoreType.DMA,
            pltpu.SemaphoreType.DMA,
        ],
        compiler_params=pltpu.CompilerParams(use_tc_tiling_on_sc=False),
    )
    def k(table_hbm, idx_hbm, pos_hbm, out_hbm, idx_v, pos_v, gbufs, obuf,
          gsem0, gsem1):
        wid = lax.axis_index("s") * NC + lax.axis_index("c")
        base = wid * b_per_w
        pltpu.sync_copy(idx_hbm.at[pl.ds(base, b_per_w)], idx_v)
        pltpu.sync_copy(pos_hbm, pos_v)
        gsems = (gsem0, gsem1)

        for p in range(2):
            pltpu.async_copy(table_hbm.at[idx_v.at[p]], gbufs.at[p], gsems[p])

        def step_body(s_, carry):
            for p in range(2):
                r = 2 * s_ + p
                gb = gbufs.at[p]
                pltpu.make_async_copy(
                    table_hbm.at[idx_v.at[r]], gb, gsems[p]).wait()

                @plsc.parallel_loop(0, S, 1, unroll=4)
                def add_body(row):
                    for j in range(D // LANES):
                        sl = pl.ds(j * LANES, LANES)
                        obuf[row, sl] = gb[row, sl] + pos_v[row, sl]

                @pl.when(r + 2 < b_per_w)
                def _():
                    pltpu.async_copy(
                        table_hbm.at[idx_v.at[r + 2]], gb, gsems[p])

                pltpu.sync_copy(obuf, out_hbm.at[base + r])
            return carry

        lax.fori_loop(0, b_per_w // 2, step_body, 0)

    return k(table, idx, pos)


def kernel(inputs, id_table, pos_table):
    return _sc_embed(id_table, inputs.astype(jnp.int32), pos_table)
